# baseline (speedup 1.0000x reference)
import functools

import jax
import jax.numpy as jnp
from jax import lax
from jax.experimental import pallas as pl
from jax.experimental.pallas import tpu as pltpu
from jax.experimental.pallas import tpu_sc as plsc

NC, NS, L = 2, 16, 16
NW = NC * NS


@functools.lru_cache(maxsize=None)
def _build(B, S, H):
    BW = B // NW
    CH = 32
    NCH = S // CH
    NG = CH // L
    T = BW * NCH
    mesh = plsc.VectorSubcoreMesh(core_axis_name="c", subcore_axis_name="s")

    @functools.partial(
        pl.kernel,
        mesh=mesh,
        compiler_params=pltpu.CompilerParams(needs_layout_passes=False),
        out_type=jax.ShapeDtypeStruct((B, S, H), jnp.float32),
        scratch_types=[
            pltpu.VMEM((BW, S), jnp.int32),
            pltpu.VMEM((BW, S), jnp.int32),
            pltpu.VMEM((H, S), jnp.float32),
            pltpu.VMEM((H // L, L), jnp.float32),
            pltpu.VMEM((H // L, L), jnp.float32),
            pltpu.VMEM((H // L, L), jnp.float32),
            pltpu.VMEM((CH, H), jnp.float32),
            pltpu.VMEM((CH, H), jnp.float32),
            pltpu.VMEM((CH, H), jnp.float32),
            pltpu.VMEM((CH, H), jnp.float32),
            pltpu.SemaphoreType.DMA,
            pltpu.SemaphoreType.DMA,
            pltpu.SemaphoreType.DMA,
            pltpu.SemaphoreType.DMA,
        ],
    )
    def sc_kernel(ids_hbm, tt_hbm, word_hbm, posT_hbm, tB_hbm, gam_hbm, bet_hbm,
                  out_hbm, ids_v, tt_v, posT_v, tB_v, gam_v, bet_v,
                  g0, g1, o0, o1, si0, si1, so0, so1):
        wid = lax.axis_index("s") * NC + lax.axis_index("c")
        b_lo = wid * BW
        pltpu.sync_copy(ids_hbm.at[pl.ds(b_lo, BW)], ids_v)
        pltpu.sync_copy(tt_hbm.at[pl.ds(b_lo, BW)], tt_v)
        pltpu.sync_copy(posT_hbm, posT_v)
        pltpu.sync_copy(tB_hbm, tB_v)
        pltpu.sync_copy(gam_hbm, gam_v)
        pltpu.sync_copy(bet_hbm, bet_v)
        rid = lax.iota(jnp.int32, L)
        zero = jnp.zeros((L,), jnp.float32)
        gbuf = (g0, g1)
        obuf = (o0, o1)
        gsem = (si0, si1)
        osem = (so0, so1)

        def idx_of(t):
            bl = t // NCH
            s0 = (t % NCH) * CH
            return bl, s0

        NSUB = 4
        RSUB = CH // NSUB

        def sub_gathers(t, k):
            bl, s0 = idx_of(t)
            return [pltpu.make_async_copy(
                word_hbm.at[ids_v.at[bl, pl.ds(s0 + j * RSUB, RSUB)]],
                gbuf[k].at[pl.ds(j * RSUB, RSUB)], gsem[k])
                for j in range(NSUB)]

        def gather_start(t, k):
            for c in sub_gathers(t, k):
                c.start()

        def gather_wait(t, k):
            for c in sub_gathers(t, k):
                c.wait()

        def copy_out(t, k):
            bl, s0 = idx_of(t)
            return pltpu.make_async_copy(
                obuf[k], out_hbm.at[b_lo + bl, pl.ds(s0, CH)], osem[k])

        gather_start(0, 0)

        def pair_body(tp, carry):
            for k in range(2):
                t = tp * 2 + k
                bl, s0 = idx_of(t)

                @pl.when(t < T - 1)
                def _():
                    gather_start(t + 1, 1 - k)

                gather_wait(t, k)

                @pl.when(t >= 2)
                def _():
                    copy_out(t - 2, k).wait()

                gv = gbuf[k]
                ov = obuf[k]
                ttf = [tt_v[bl, pl.ds(s0 + g * L, L)].astype(jnp.float32)
                       for g in range(NG)]

                def p1(h, c):
                    sms, sqs = c
                    hs = jnp.full((L,), h, jnp.int32)
                    tb = plsc.load_gather(tB_v, [hs >> 4, hs & 15])
                    n_sms, n_sqs = [], []
                    for g in range(NG):
                        w = plsc.load_gather(gv, [rid + g * L, hs])
                        p = posT_v[h, pl.ds(s0 + g * L, L)]
                        cv = w + p + ttf[g] * tb
                        plsc.store_scatter(ov, [rid + g * L, hs], cv)
                        n_sms.append(sms[g] + cv)
                        n_sqs.append(sqs[g] + cv * cv)
                    return (tuple(n_sms), tuple(n_sqs))

                sms, sqs = lax.fori_loop(
                    0, H, p1, (tuple([zero] * NG), tuple([zero] * NG)),
                    unroll=4)

                means, rstds = [], []
                for g in range(NG):
                    mean = sms[g] * (1.0 / H)
                    var = sqs[g] * (1.0 / H) - mean * mean
                    x = var + 1e-12
                    i = plsc.bitcast(x, jnp.int32)
                    y = plsc.bitcast(
                        jnp.int32(0x5F3759DF) - (i >> 1), jnp.float32)
                    for _ in range(3):
                        y = y * (1.5 - 0.5 * x * y * y)
                    means.append(mean)
                    rstds.append(y)

                def p2(h, c):
                    hs = jnp.full((L,), h, jnp.int32)
                    ga = plsc.load_gather(gam_v, [hs >> 4, hs & 15])
                    be = plsc.load_gather(bet_v, [hs >> 4, hs & 15])
                    for g in range(NG):
                        cv = plsc.load_gather(ov, [rid + g * L, hs])
                        nv = (cv - means[g]) * rstds[g]
                        plsc.store_scatter(ov, [rid + g * L, hs],
                                           nv * ga + be)
                    return c

                lax.fori_loop(0, H, p2, 0, unroll=4)
                copy_out(t, k).start()
            return carry

        lax.fori_loop(0, T // 2, pair_body, 0)
        copy_out(T - 2, 0).wait()
        copy_out(T - 1, 1).wait()

    return sc_kernel


def kernel(input_ids, token_type_ids, word_emb, pos_emb, type_emb, gamma, beta):
    B, S = input_ids.shape
    H = word_emb.shape[1]
    ids = input_ids.astype(jnp.int32)
    tt = token_type_ids.astype(jnp.int32)
    posT = pos_emb[:S].astype(jnp.float32).T + type_emb[0][:, None]
    tB = (type_emb[1] - type_emb[0]).reshape(H // 16, 16)
    fn = _build(B, S, H)
    return fn(ids, tt, word_emb.astype(jnp.float32), posT, tB,
              gamma.astype(jnp.float32).reshape(H // 16, 16),
              beta.astype(jnp.float32).reshape(H // 16, 16))


# speedup vs baseline: 5.6784x; 5.6784x over previous
import functools

import jax
import jax.numpy as jnp
from jax import lax
from jax.experimental import pallas as pl
from jax.experimental.pallas import tpu as pltpu
from jax.experimental.pallas import tpu_sc as plsc

NC, NS, L = 2, 16, 16
NW = NC * NS
NBUF = 4


@functools.lru_cache(maxsize=None)
def _build(B, S, H):
    SW = S // NW
    HPC = H // L
    T = B
    mesh = plsc.VectorSubcoreMesh(core_axis_name="c", subcore_axis_name="s")

    @functools.partial(
        pl.kernel,
        mesh=mesh,
        compiler_params=pltpu.CompilerParams(needs_layout_passes=False),
        out_type=jax.ShapeDtypeStruct((B, S, H), jnp.float32),
        scratch_types=(
            [pltpu.VMEM((B * SW,), jnp.int32),
             pltpu.VMEM((B * SW,), jnp.int32),
             pltpu.VMEM((SW, H), jnp.float32),
             pltpu.VMEM((H,), jnp.float32),
             pltpu.VMEM((H,), jnp.float32),
             pltpu.VMEM((H,), jnp.float32)]
            + [pltpu.VMEM((SW, H), jnp.float32) for _ in range(2 * NBUF)]
            + [pltpu.SemaphoreType.DMA for _ in range(2 * NBUF)]
        ),
    )
    def sc_kernel(ids_hbm, tt_hbm, word_hbm, pos_hbm, tB_hbm, gam_hbm, bet_hbm,
                  out_hbm, ids_v, tt_v, pos_v, tB_v, gam_v, bet_v, *bufs):
        gbuf = bufs[0:NBUF]
        obuf = bufs[NBUF:2 * NBUF]
        gsem = bufs[2 * NBUF:3 * NBUF]
        osem = bufs[3 * NBUF:4 * NBUF]
        wid = lax.axis_index("s") * NC + lax.axis_index("c")
        s_lo = wid * SW
        pltpu.sync_copy(ids_hbm.at[wid], ids_v)
        pltpu.sync_copy(tt_hbm.at[wid], tt_v)
        pltpu.sync_copy(pos_hbm.at[wid], pos_v)
        pltpu.sync_copy(tB_hbm, tB_v)
        pltpu.sync_copy(gam_hbm, gam_v)
        pltpu.sync_copy(bet_hbm, bet_v)

        def gather(t, k):
            return pltpu.make_async_copy(
                word_hbm.at[ids_v.at[pl.ds(t * SW, SW)]], gbuf[k], gsem[k])

        def out_copy(t, k):
            return pltpu.make_async_copy(
                obuf[k], out_hbm.at[t, pl.ds(s_lo, SW)], osem[k])

        gather(0, 0).start()
        gather(1, 1).start()

        def quad_body(tq, carry):
            for k in range(NBUF):
                t = tq * NBUF + k

                @pl.when(t >= 2)
                def _():
                    out_copy(t - 2, (k - 2) % NBUF).wait()

                @pl.when(t < T - 2)
                def _():
                    gather(t + 2, (k + 2) % NBUF).start()

                gather(t, k).wait()

                gv = gbuf[k]
                ov = obuf[k]
                ttv = tt_v[pl.ds(t * SW, SW)].astype(jnp.float32)
                tBp = [tB_v[pl.ds(p * L, L)] for p in range(HPC)]
                gap = [gam_v[pl.ds(p * L, L)] for p in range(HPC)]
                bep = [bet_v[pl.ds(p * L, L)] for p in range(HPC)]

                for i in range(SW):
                    ttf = ttv[i]
                    c = [gv[i, pl.ds(p * L, L)] + pos_v[i, pl.ds(p * L, L)]
                         + ttf * tBp[p] for p in range(HPC)]
                    s = c
                    while len(s) > 1:
                        s = [s[2 * j] + s[2 * j + 1] for j in range(len(s) // 2)]
                    q = [cp * cp for cp in c]
                    while len(q) > 1:
                        q = [q[2 * j] + q[2 * j + 1] for j in range(len(q) // 2)]
                    s1 = jnp.full((L,), jnp.sum(s[0]))
                    s2 = jnp.full((L,), jnp.sum(q[0]))
                    mean = s1 * (1.0 / H)
                    var = s2 * (1.0 / H) - mean * mean
                    x = var + 1e-12
                    iv = plsc.bitcast(x, jnp.int32)
                    y = plsc.bitcast(
                        jnp.int32(0x5F3759DF) - (iv >> 1), jnp.float32)
                    for _ in range(3):
                        y = y * (1.5 - 0.5 * x * y * y)
                    for p in range(HPC):
                        ov[i, pl.ds(p * L, L)] = (
                            (c[p] - mean) * (y * gap[p]) + bep[p])

                out_copy(t, k).start()
            return carry

        lax.fori_loop(0, T // NBUF, quad_body, 0)
        out_copy(T - 2, (T - 2) % NBUF).wait()
        out_copy(T - 1, (T - 1) % NBUF).wait()

    return sc_kernel


def kernel(input_ids, token_type_ids, word_emb, pos_emb, type_emb, gamma, beta):
    B, S = input_ids.shape
    H = word_emb.shape[1]
    SW = S // NW
    ids = (input_ids.astype(jnp.int32).T.reshape(NW, SW, B)
           .transpose(0, 2, 1).reshape(NW, B * SW))
    tt = (token_type_ids.astype(jnp.int32).T.reshape(NW, SW, B)
          .transpose(0, 2, 1).reshape(NW, B * SW))
    pos = (pos_emb[:S].astype(jnp.float32)
           + type_emb[0][None, :]).reshape(NW, SW, H)
    tB = type_emb[1] - type_emb[0]
    fn = _build(B, S, H)
    return fn(ids, tt, word_emb.astype(jnp.float32), pos, tB,
              gamma.astype(jnp.float32), beta.astype(jnp.float32))
